# Initial kernel scaffold; baseline (speedup 1.0000x reference)
import jax
import jax.numpy as jnp
from jax.experimental import pallas as pl


def kernel(adj0_indices, adj0_values, adj1_indices, adj1_values, adj2_indices, adj2_values, adj3_indices, adj3_values, feat0, feat1, feat2, fc1_W1, fc1_b1, fc1_W2, fc1_b2, fc2_W1, fc2_b1, fc2_W2, fc2_b2):
    raise NotImplementedError("write your pallas kernel here")


# speedup vs baseline: 3.6684x; 3.6684x over previous
import functools

import jax
import jax.numpy as jnp
from jax import lax
from jax.experimental import pallas as pl
from jax.experimental.pallas import tpu as pltpu
from jax.experimental.pallas import tpu_sc as plsc

_NUM_CORES = 2
_NUM_SUBCORES = 16
_NW = _NUM_CORES * _NUM_SUBCORES

_CHUNK = 80


def _make_spmm(n, e, d):
  epw = e // _NW
  nchunk = epw // _CHUNK
  rows_per_tile = 128 * ((n + 128 * _NUM_SUBCORES - 1) // (128 * _NUM_SUBCORES))
  npad = rows_per_tile * _NUM_SUBCORES
  zr = 128
  zsteps = rows_per_tile // zr

  mesh = plsc.VectorSubcoreMesh(core_axis_name="c", subcore_axis_name="s")

  @functools.partial(
      pl.kernel,
      out_type=jax.ShapeDtypeStruct((_NUM_CORES, npad, d), jnp.float32),
      mesh=mesh,
      scratch_types=[
          pltpu.VMEM_SHARED((npad, d), jnp.float32),
          pltpu.VMEM((_CHUNK,), jnp.int32),
          pltpu.VMEM((_CHUNK,), jnp.int32),
          pltpu.VMEM((_CHUNK,), jnp.float32),
          pltpu.VMEM((_CHUNK, d), jnp.float32),
          pltpu.VMEM((zr, d), jnp.float32),
          pltpu.SemaphoreType.DMA,
          pltpu.SemaphoreType.DMA,
      ],
  )
  def spmm(row_hbm, col_hbm, val_hbm, m_hbm, out_hbm,
           acc, colv, rowv, valv, rows, zbuf, gsem, ssem):
    c = lax.axis_index("c")
    s = lax.axis_index("s")
    wid = s * _NUM_CORES + c
    base = wid * epw

    def zrow(i, carry):
      for j in range(d // 16):
        zbuf[i, pl.ds(j * 16, 16)] = jnp.zeros((16,), jnp.float32)
      return carry
    lax.fori_loop(0, zr, zrow, 0)
    for t in range(zsteps):
      pltpu.sync_copy(zbuf, acc.at[pl.ds(s * rows_per_tile + t * zr, zr)])
    plsc.subcore_barrier()

    def chunk(i, carry):
      off = base + i * _CHUNK
      pltpu.sync_copy(col_hbm.at[pl.ds(off, _CHUNK)], colv)
      pltpu.sync_copy(row_hbm.at[pl.ds(off, _CHUNK)], rowv)
      pltpu.sync_copy(val_hbm.at[pl.ds(off, _CHUNK)], valv)
      pltpu.async_copy(m_hbm.at[colv], rows, gsem).wait()

      def scale_group(g, carry2):
        vv = valv[pl.ds(g * 16, 16)]
        for j in range(16):
          v = vv[j]
          k = g * 16 + j
          for jj in range(d // 16):
            sl = pl.ds(jj * 16, 16)
            rows[k, sl] = rows[k, sl] * v
        return carry2
      lax.fori_loop(0, _CHUNK // 16, scale_group, 0)

      pltpu.async_copy(rows, acc.at[rowv], ssem, add=True).wait()
      return carry
    lax.fori_loop(0, nchunk, chunk, 0)

    plsc.subcore_barrier()
    for t in range(zsteps):
      sl = pl.ds(s * rows_per_tile + t * zr, zr)
      pltpu.sync_copy(acc.at[sl], out_hbm.at[c, sl])

  return spmm


_ROWS_BLK = 1000


def _mlp_body(nin, npost, want_x, x_ref, *refs):
  pres = refs[:nin]
  w1_ref, b1_ref, w2_ref, b2_ref = refs[nin:nin + 4]
  posts = refs[nin + 4:nin + 4 + npost]
  outs = refs[nin + 4 + npost:]
  x = x_ref[...]
  for p in pres:
    x = x + p[...]
  dn = (((1,), (0,)), ((), ()))
  h = lax.dot_general(x, w1_ref[...], dn,
                      preferred_element_type=jnp.float32,
                      precision=lax.Precision.HIGHEST)
  h = jnp.maximum(h + b1_ref[...], 0.0)
  y = lax.dot_general(h, w2_ref[...], dn,
                      preferred_element_type=jnp.float32,
                      precision=lax.Precision.HIGHEST)
  y = y + b2_ref[...]
  for p in posts:
    y = y + p[...]
  outs[0][...] = y
  if want_x:
    outs[1][...] = x


def _mlp(x, w1, b1, w2, b2, pre=(), post=(), want_x=False):
  n, d = x.shape
  grid = (n // _ROWS_BLK,)
  row_spec = pl.BlockSpec((_ROWS_BLK, d), lambda i: (i, 0))
  w_spec = pl.BlockSpec((d, d), lambda i: (0, 0))
  b_spec = pl.BlockSpec((1, d), lambda i: (0, 0))
  in_specs = ([row_spec] + [row_spec] * len(pre) + [w_spec, b_spec, w_spec,
                                                   b_spec]
              + [row_spec] * len(post))
  out_shape = [jax.ShapeDtypeStruct((n, d), jnp.float32)]
  out_specs = [row_spec]
  if want_x:
    out_shape.append(jax.ShapeDtypeStruct((n, d), jnp.float32))
    out_specs.append(row_spec)
  fn = pl.pallas_call(
      functools.partial(_mlp_body, len(pre), len(post), want_x),
      grid=grid,
      in_specs=in_specs,
      out_specs=out_specs,
      out_shape=out_shape,
  )
  res = fn(x, *pre, w1, b1.reshape(1, d), w2, b2.reshape(1, d), *post)
  if want_x:
    return res[0], res[1]
  return res[0]


def kernel(adj0_indices, adj0_values, adj1_indices, adj1_values,
           adj2_indices, adj2_values, adj3_indices, adj3_values,
           feat0, feat1, feat2,
           fc1_W1, fc1_b1, fc1_W2, fc1_b2,
           fc2_W1, fc2_b1, fc2_W2, fc2_b2):
  n, d = feat0.shape
  e = adj0_values.shape[0]
  spmm = _make_spmm(n, e, d)

  def do_spmm(idx, vals, m):
    part = spmm(idx[0], idx[1], vals, m)
    return part[0, :n], part[1, :n]

  mlp1 = lambda i, x, **kw: _mlp(x, fc1_W1[i], fc1_b1[i], fc1_W2[i],
                                 fc1_b2[i], **kw)
  mlp2 = lambda i, x, **kw: _mlp(x, fc2_W1[i], fc2_b1[i], fc2_W2[i],
                                 fc2_b2[i], **kw)

  b3 = mlp1(2, feat2)
  m3 = mlp1(3, feat2)
  p3 = do_spmm(adj3_indices, adj3_values, m3)
  b2 = mlp1(1, feat1)
  m2 = mlp2(3, b3, pre=p3)
  p2 = do_spmm(adj2_indices, adj2_values, m2)
  b1, x2 = mlp2(1, b2, pre=p2, want_x=True)
  m1 = mlp2(2, x2)
  p1 = do_spmm(adj1_indices, adj1_values, m1)
  m0 = mlp2(0, b1, pre=p1)
  p0 = do_spmm(adj0_indices, adj0_values, m0)
  out = mlp1(0, feat0, post=p0)
  return out


# speedup vs baseline: 9.0186x; 2.4584x over previous
import functools

import jax
import jax.numpy as jnp
from jax import lax
from jax.experimental import pallas as pl
from jax.experimental.pallas import tpu as pltpu
from jax.experimental.pallas import tpu_sc as plsc

_NUM_CORES = 2
_NUM_SUBCORES = 16
_NW = _NUM_CORES * _NUM_SUBCORES

_CHUNK = 80


def _make_spmm(n, e, d):
  epw = e // _NW
  nchunk = epw // _CHUNK
  rows_per_tile = 128 * ((n + 128 * _NUM_SUBCORES - 1) // (128 * _NUM_SUBCORES))
  npad = rows_per_tile * _NUM_SUBCORES

  mesh = plsc.VectorSubcoreMesh(core_axis_name="c", subcore_axis_name="s")

  @functools.partial(
      pl.kernel,
      out_type=jax.ShapeDtypeStruct((_NUM_CORES, npad, d), jnp.float32),
      mesh=mesh,
      scratch_types=[
          pltpu.VMEM_SHARED((npad, d), jnp.float32),
          pltpu.VMEM((4, _CHUNK), jnp.int32),
          pltpu.VMEM((4, _CHUNK), jnp.int32),
          pltpu.VMEM((4, _CHUNK), jnp.float32),
          pltpu.VMEM((4, _CHUNK, d), jnp.float32),
          pltpu.SemaphoreType.DMA,
          pltpu.SemaphoreType.DMA,
          pltpu.SemaphoreType.DMA,
      ],
  )
  def spmm(row_hbm, col_hbm, val_hbm, m_hbm, z_hbm, out_hbm,
           acc, colb, rowb, valb, rows, isem, gsem, ssem):
    c = lax.axis_index("c")
    s = lax.axis_index("s")
    wid = s * _NUM_CORES + c
    base = wid * epw

    pltpu.sync_copy(z_hbm, acc.at[pl.ds(s * rows_per_tile, rows_per_tile)])
    plsc.subcore_barrier()

    def idx_copies(i, b):
      off = base + i * _CHUNK
      return (
          (col_hbm.at[pl.ds(off, _CHUNK)], colb.at[b]),
          (row_hbm.at[wid, i], rowb.at[b]),
          (val_hbm.at[pl.ds(off, _CHUNK)], valb.at[b]),
      )

    def start_idx(i, b):
      for src, dst in idx_copies(i, b):
        pltpu.async_copy(src, dst, isem)

    def wait_idx(i, b):
      for src, dst in idx_copies(i, b):
        pltpu.make_async_copy(src, dst, isem).wait()

    def start_gather(i, b):
      pltpu.async_copy(m_hbm.at[colb.at[b]], rows.at[b], gsem)

    def wait_gather(i, b):
      pltpu.make_async_copy(m_hbm.at[colb.at[b]], rows.at[b], gsem).wait()

    def start_scatter(i, b):
      pltpu.async_copy(rows.at[b], acc.at[rowb.at[b]], ssem, add=True)

    def wait_scatter(i, b):
      pltpu.make_async_copy(rows.at[b], acc.at[rowb.at[b]], ssem).wait()

    def scale(i, b):
      def grp(g, carry):
        vv = valb[b, pl.ds(g * 16, 16)]
        for j in range(16):
          v = vv[j]
          k = g * 16 + j
          for jj in range(d // 16):
            sl = pl.ds(jj * 16, 16)
            rows[b, k, sl] = rows[b, k, sl] * v
        return carry
      lax.fori_loop(0, _CHUNK // 16, grp, 0)

    def step(i, b):
      @pl.when(i >= 2)
      def _():
        wait_scatter(i - 2, (b + 2) % 4)

      @pl.when(i + 1 < nchunk)
      def _():
        wait_idx(i + 1, (b + 1) % 4)
        start_gather(i + 1, (b + 1) % 4)

      @pl.when(i + 2 < nchunk)
      def _():
        start_idx(i + 2, (b + 2) % 4)

      wait_gather(i, b)
      scale(i, b)
      start_scatter(i, b)

    start_idx(0, 0)
    start_idx(1, 1)
    wait_idx(0, 0)
    start_gather(0, 0)

    n_main = nchunk - (nchunk % 4)

    def quad(q, carry):
      for b in range(4):
        step(4 * q + b, b)
      return carry
    lax.fori_loop(0, n_main // 4, quad, 0)

    for i in range(n_main, nchunk):
      b = i % 4
      wait_scatter(i - 2, (b + 2) % 4)
      if i + 1 < nchunk:
        wait_idx(i + 1, (b + 1) % 4)
        start_gather(i + 1, (b + 1) % 4)
      wait_gather(i, b)
      scale(i, b)
      start_scatter(i, b)

    wait_scatter(nchunk - 2, (nchunk - 2) % 4)
    wait_scatter(nchunk - 1, (nchunk - 1) % 4)

    plsc.subcore_barrier()
    sl = pl.ds(s * rows_per_tile, rows_per_tile)
    pltpu.sync_copy(acc.at[sl], out_hbm.at[c, sl])

  return spmm


_ROWS_BLK = 1000


def _mlp_body(nin, npost, want_x, x_ref, *refs):
  pres = refs[:nin]
  w1_ref, b1_ref, w2_ref, b2_ref = refs[nin:nin + 4]
  posts = refs[nin + 4:nin + 4 + npost]
  outs = refs[nin + 4 + npost:]
  x = x_ref[...]
  for p in pres:
    x = x + p[...]
  dn = (((1,), (0,)), ((), ()))
  h = lax.dot_general(x, w1_ref[...], dn,
                      preferred_element_type=jnp.float32,
                      precision=lax.Precision.HIGHEST)
  h = jnp.maximum(h + b1_ref[...], 0.0)
  y = lax.dot_general(h, w2_ref[...], dn,
                      preferred_element_type=jnp.float32,
                      precision=lax.Precision.HIGHEST)
  y = y + b2_ref[...]
  for p in posts:
    y = y + p[...]
  outs[0][...] = y
  if want_x:
    outs[1][...] = x


def _mlp(x, w1, b1, w2, b2, pre=(), post=(), want_x=False):
  n, d = x.shape
  grid = (n // _ROWS_BLK,)
  row_spec = pl.BlockSpec((_ROWS_BLK, d), lambda i: (i, 0))
  w_spec = pl.BlockSpec((d, d), lambda i: (0, 0))
  b_spec = pl.BlockSpec((1, d), lambda i: (0, 0))
  in_specs = ([row_spec] + [row_spec] * len(pre) + [w_spec, b_spec, w_spec,
                                                   b_spec]
              + [row_spec] * len(post))
  out_shape = [jax.ShapeDtypeStruct((n, d), jnp.float32)]
  out_specs = [row_spec]
  if want_x:
    out_shape.append(jax.ShapeDtypeStruct((n, d), jnp.float32))
    out_specs.append(row_spec)
  fn = pl.pallas_call(
      functools.partial(_mlp_body, len(pre), len(post), want_x),
      grid=grid,
      in_specs=in_specs,
      out_specs=out_specs,
      out_shape=out_shape,
  )
  res = fn(x, *pre, w1, b1.reshape(1, d), w2, b2.reshape(1, d), *post)
  if want_x:
    return res[0], res[1]
  return res[0]


def kernel(adj0_indices, adj0_values, adj1_indices, adj1_values,
           adj2_indices, adj2_values, adj3_indices, adj3_values,
           feat0, feat1, feat2,
           fc1_W1, fc1_b1, fc1_W2, fc1_b2,
           fc2_W1, fc2_b1, fc2_W2, fc2_b2):
  n, d = feat0.shape
  e = adj0_values.shape[0]
  spmm = _make_spmm(n, e, d)

  epw = e // _NW
  nchunk = epw // _CHUNK
  rows_per_tile = 128 * ((n + 128 * _NUM_SUBCORES - 1) // (128 * _NUM_SUBCORES))
  zeros = jnp.zeros((rows_per_tile, d), jnp.float32)

  def do_spmm(idx, vals, m):
    row3d = idx[0].reshape(_NW, nchunk, _CHUNK)
    part = spmm(row3d, idx[1], vals, m, zeros)
    return part[0, :n], part[1, :n]

  mlp1 = lambda i, x, **kw: _mlp(x, fc1_W1[i], fc1_b1[i], fc1_W2[i],
                                 fc1_b2[i], **kw)
  mlp2 = lambda i, x, **kw: _mlp(x, fc2_W1[i], fc2_b1[i], fc2_W2[i],
                                 fc2_b2[i], **kw)

  b3 = mlp1(2, feat2)
  m3 = mlp1(3, feat2)
  p3 = do_spmm(adj3_indices, adj3_values, m3)
  b2 = mlp1(1, feat1)
  m2 = mlp2(3, b3, pre=p3)
  p2 = do_spmm(adj2_indices, adj2_values, m2)
  b1, x2 = mlp2(1, b2, pre=p2, want_x=True)
  m1 = mlp2(2, x2)
  p1 = do_spmm(adj1_indices, adj1_values, m1)
  m0 = mlp2(0, b1, pre=p1)
  p0 = do_spmm(adj0_indices, adj0_values, m0)
  out = mlp1(0, feat0, post=p0)
  return out
